# baseline (speedup 1.0000x reference)
import functools

import jax
import jax.numpy as jnp
from jax import lax
from jax.experimental import pallas as pl
from jax.experimental.pallas import tpu as pltpu
from jax.experimental.pallas import tpu_sc as plsc

KNB = 20
NPTS = 2048
NBATCH = 8
RBLK = 256
NPB = 128
NBLK = NPTS // NPB
EPS = 1e-5


def _lrelu(t):
    return jnp.where(t > 0, t, 0.2 * t)


def _nt(a, b):
    return lax.dot_general(a, b, (((1,), (1,)), ((), ())),
                           preferred_element_type=jnp.float32,
                           precision=lax.Precision.HIGHEST)


def _cast16(t):
    return t.astype(jnp.bfloat16)


def _nt_bf16(a, b):
    return lax.dot_general(a.astype(jnp.bfloat16), b.astype(jnp.bfloat16),
                           (((1,), (1,)), ((), ())),
                           preferred_element_type=jnp.float32)


def _knn_body(xb_ref, xa_ref, out_ref, pair_ref):
    b = pl.program_id(0)
    xb = xb_ref[0]
    xa = xa_ref[0]
    xxa = _nt(jnp.ones((1, xa.shape[1]), jnp.float32), xa * xa)
    xxb = jnp.sum(xb * xb, axis=1, keepdims=True)
    g = _nt_bf16(xb, xa)
    pair_ref[...] = 2.0 * g - xxb - xxa
    iota = lax.broadcasted_iota(jnp.int32, (RBLK, NPTS), 1)
    kio = lax.broadcasted_iota(jnp.int32, (RBLK, KNB), 1)

    def body(i, amin_prev):
        p = jnp.where(iota == amin_prev, -jnp.inf, pair_ref[...])
        pair_ref[...] = p
        vmax = jnp.max(p, axis=1, keepdims=True)
        cand = jnp.where(p == vmax, iota, NPTS)
        amin = jnp.min(cand, axis=1, keepdims=True)
        out_ref[0] = jnp.where(kio == i,
                               jnp.broadcast_to(amin + b * NPTS, (RBLK, KNB)),
                               out_ref[0])
        return amin

    lax.fori_loop(0, KNB, body, jnp.full((RBLK, 1), -1, jnp.int32))


def _knn_topk(xt):
    B, N, C = xt.shape
    grid = (B, N // RBLK)
    return pl.pallas_call(
        _knn_body,
        grid=grid,
        in_specs=[
            pl.BlockSpec((1, RBLK, C), lambda b, r: (b, r, 0)),
            pl.BlockSpec((1, N, C), lambda b, r: (b, 0, 0)),
        ],
        out_specs=pl.BlockSpec((1, RBLK, KNB), lambda b, r: (b, r, 0)),
        out_shape=jax.ShapeDtypeStruct((B, N, KNB), jnp.int32),
        scratch_shapes=[pltpu.VMEM((RBLK, NPTS), jnp.float32)],
    )(xt, xt)


def _bmm_body(h_ref, w_ref, o_ref):
    o_ref[0] = _nt_bf16(h_ref[0], w_ref[...])


def _bmm(h, w):
    B, N, Cin = h.shape
    Cout = w.shape[0]
    return pl.pallas_call(
        _bmm_body,
        grid=(B,),
        in_specs=[
            pl.BlockSpec((1, N, Cin), lambda b: (b, 0, 0)),
            pl.BlockSpec((Cout, Cin), lambda b: (0, 0)),
        ],
        out_specs=pl.BlockSpec((1, N, Cout), lambda b: (b, 0, 0)),
        out_shape=jax.ShapeDtypeStruct((B, N, Cout), jnp.float32),
    )(h, w)


def _sc_gather(table, idx):
    R, C = table.shape
    (M,) = idx.shape
    info = plsc.get_sparse_core_info()
    nw = info.num_cores * info.num_subcores
    m_per_w = M // nw
    ch = 256
    n_ch = m_per_w // ch
    assert m_per_w % ch == 0 and n_ch % 2 == 0
    mesh = plsc.VectorSubcoreMesh(core_axis_name="c", subcore_axis_name="s")

    @functools.partial(
        pl.kernel, mesh=mesh,
        out_type=jax.ShapeDtypeStruct((M, C), jnp.float32),
        scratch_types=[
            pltpu.VMEM((ch,), jnp.int32),
            pltpu.VMEM((ch,), jnp.int32),
            pltpu.VMEM((ch, C), jnp.float32),
            pltpu.VMEM((ch, C), jnp.float32),
            pltpu.SemaphoreType.DMA,
            pltpu.SemaphoreType.DMA,
        ],
    )
    def k(table_hbm, idx_hbm, out_hbm, idx0, idx1, rows0, rows1, s0, s1):
        wid = lax.axis_index("s") * info.num_cores + lax.axis_index("c")
        base = wid * m_per_w

        def issue(j, idx_v, rows_v, sem):
            off = base + j * ch
            pltpu.sync_copy(idx_hbm.at[pl.ds(off, ch)], idx_v)
            pltpu.async_copy(table_hbm.at[idx_v], rows_v, sem)

        def drain(j, idx_v, rows_v, sem):
            pltpu.make_async_copy(table_hbm.at[idx_v], rows_v, sem).wait()
            pltpu.sync_copy(rows_v, out_hbm.at[pl.ds(base + j * ch, ch)])

        issue(0, idx0, rows0, s0)

        def step2(jj, _):
            j = jj * 2
            issue(j + 1, idx1, rows1, s1)
            drain(j, idx0, rows0, s0)

            @pl.when(jj + 1 < n_ch // 2)
            def _():
                issue(j + 2, idx0, rows0, s0)

            drain(j + 1, idx1, rows1, s1)
            return 0

        lax.fori_loop(0, n_ch // 2, step2, 0)

    return k(table, idx)


def _acc_rows(out_ref, s, ss):
    rio = lax.broadcasted_iota(jnp.int32, out_ref.shape, 0)
    upd = jnp.where(rio == 0, jnp.broadcast_to(s, out_ref.shape), 0.0)
    upd = upd + jnp.where(rio == 1, jnp.broadcast_to(ss, out_ref.shape), 0.0)
    out_ref[...] += upd


def _h1pre(nbr_ref, z_ref, w1_ref):
    cv = z_ref.shape[-1]
    nb = nbr_ref[0, 0][:, :cv]
    ctr = jnp.broadcast_to(z_ref[0, 0][:, None, :], (NPB, KNB, cv))
    ctr = ctr.reshape(NPB * KNB, cv)
    d16 = _cast16(nb - ctr)
    c16 = _cast16(ctr)
    parts = [d16, c16]
    if 2 * cv < 128:
        parts.append(jnp.zeros((NPB * KNB, 128 - 2 * cv), d16.dtype))
    feat = jnp.concatenate(parts, axis=1)
    return _nt_bf16(feat, w1_ref[...])


def _ec_stats1_body(nbr_ref, z_ref, w1_ref, out_ref):
    @pl.when(jnp.logical_and(pl.program_id(0) == 0, pl.program_id(1) == 0))
    def _():
        out_ref[...] = jnp.zeros_like(out_ref)

    hh = _h1pre(nbr_ref, z_ref, w1_ref)
    s = jnp.sum(hh, axis=0, keepdims=True)
    ss = jnp.sum(hh * hh, axis=0, keepdims=True)
    _acc_rows(out_ref, s, ss)


def _ec_stats2_body(nbr_ref, z_ref, w1_ref, sc1_ref, w2_ref, out_ref):
    @pl.when(jnp.logical_and(pl.program_id(0) == 0, pl.program_id(1) == 0))
    def _():
        out_ref[...] = jnp.zeros_like(out_ref)

    hh = _h1pre(nbr_ref, z_ref, w1_ref)
    h1 = _lrelu(hh * sc1_ref[0:1, :] + sc1_ref[1:2, :])
    g = _nt_bf16(h1, w2_ref[...])
    s = jnp.sum(g, axis=0, keepdims=True)
    ss = jnp.sum(g * g, axis=0, keepdims=True)
    _acc_rows(out_ref, s, ss)


def _ec_apply_body(nbr_ref, z_ref, w1_ref, sc1_ref, w2_ref, sc2_ref, out_ref,
                   *, dup_out):
    c2 = w2_ref.shape[0]
    hh = _h1pre(nbr_ref, z_ref, w1_ref)
    h1 = _lrelu(hh * sc1_ref[0:1, :] + sc1_ref[1:2, :])
    g = _nt_bf16(h1, w2_ref[...])
    t = _lrelu(g * sc2_ref[0:1, :] + sc2_ref[1:2, :])
    m = jnp.max(t.reshape(NPB, KNB, c2), axis=1)
    if dup_out:
        m = jnp.concatenate([m, m], axis=1)
    out_ref[0, 0] = m


def _ec_grid_call(body, nbr4, z4, extras, out_shape, out_spec):
    B = z4.shape[0]
    c1 = z4.shape[-1]
    cg = nbr4.shape[-1]
    in_specs = [
        pl.BlockSpec((1, 1, NPB * KNB, cg), lambda b, r: (b, r, 0, 0)),
        pl.BlockSpec((1, 1, NPB, c1), lambda b, r: (b, r, 0, 0)),
    ]
    args = [nbr4, z4]
    for e in extras:
        in_specs.append(
            pl.BlockSpec(e.shape, (lambda nd: lambda b, r: (0,) * nd)(e.ndim)))
        args.append(e)
    return pl.pallas_call(
        body,
        grid=(B, NBLK),
        in_specs=in_specs,
        out_specs=out_spec,
        out_shape=out_shape,
    )(*args)


def _stats_to_scale(st, g, b, cnt):
    mean = st[0] / cnt
    var = st[1] / cnt - mean * mean
    s = g / jnp.sqrt(var + EPS)
    o = b - mean * s
    pad = jnp.zeros((6, s.shape[0]), jnp.float32)
    return jnp.concatenate([s[None, :], o[None, :], pad], axis=0)


def _edge_conv(xr, gidx_flat, w1p, w2, g1, b1, g2, b2, dup_out=False):
    B, N, cv = xr.shape
    c1 = w1p.shape[0]
    c2 = w2.shape[0]
    xp = xr if cv == 128 else jnp.pad(xr, ((0, 0), (0, 0), (0, 128 - cv)))
    nbr = _sc_gather(xp.reshape(B * N, 128), gidx_flat)
    nbr4 = nbr.reshape(B, NBLK, NPB * KNB, 128)
    z4 = xr.reshape(B, NBLK, NPB, cv)
    cnt = B * N * KNB
    st1 = _ec_grid_call(
        _ec_stats1_body, nbr4, z4, [w1p],
        jax.ShapeDtypeStruct((8, c1), jnp.float32),
        pl.BlockSpec((8, c1), lambda b, r: (0, 0)))
    sc1 = _stats_to_scale(st1, g1, b1, cnt)
    st2 = _ec_grid_call(
        _ec_stats2_body, nbr4, z4, [w1p, sc1, w2],
        jax.ShapeDtypeStruct((8, c2), jnp.float32),
        pl.BlockSpec((8, c2), lambda b, r: (0, 0)))
    sc2 = _stats_to_scale(st2, g2, b2, cnt)
    c_out = 2 * c2 if dup_out else c2
    out = _ec_grid_call(
        functools.partial(_ec_apply_body, dup_out=dup_out), nbr4, z4,
        [w1p, sc1, w2, sc2],
        jax.ShapeDtypeStruct((B, NBLK, NPB, c_out), jnp.float32),
        pl.BlockSpec((1, 1, NPB, c_out), lambda b, r: (b, r, 0, 0)))
    return out.reshape(B, N, c_out)


def _lin_stats_body(h_ref, w_ref, out_ref):
    @pl.when(pl.program_id(0) == 0)
    def _():
        out_ref[...] = jnp.zeros_like(out_ref)

    pre = _nt_bf16(h_ref[0], w_ref[...])
    s = jnp.sum(pre, axis=0, keepdims=True)
    ss = jnp.sum(pre * pre, axis=0, keepdims=True)
    _acc_rows(out_ref, s, ss)


def _lin_stats(h, w):
    B, N, cin = h.shape
    cout = w.shape[0]
    return pl.pallas_call(
        _lin_stats_body,
        grid=(B,),
        in_specs=[
            pl.BlockSpec((1, N, cin), lambda b: (b, 0, 0)),
            pl.BlockSpec((cout, cin), lambda b: (0, 0)),
        ],
        out_specs=pl.BlockSpec((8, cout), lambda b: (0, 0)),
        out_shape=jax.ShapeDtypeStruct((8, cout), jnp.float32),
    )(h, w)


def _lin_max_body(h_ref, w_ref, sc_ref, out_ref):
    b = pl.program_id(0)
    pre = _nt_bf16(h_ref[0], w_ref[...])
    act = _lrelu(pre * sc_ref[0:1, :] + sc_ref[1:2, :])
    row = jnp.max(act, axis=0, keepdims=True)
    rio = lax.broadcasted_iota(jnp.int32, out_ref.shape, 0)
    out_ref[...] = jnp.where(rio == b, jnp.broadcast_to(row, out_ref.shape),
                             out_ref[...])


def _lin_max(h, w, sc):
    B, N, cin = h.shape
    cout = w.shape[0]
    return pl.pallas_call(
        _lin_max_body,
        grid=(B,),
        in_specs=[
            pl.BlockSpec((1, N, cin), lambda b: (b, 0, 0)),
            pl.BlockSpec((cout, cin), lambda b: (0, 0)),
            pl.BlockSpec((8, cout), lambda b: (0, 0)),
        ],
        out_specs=pl.BlockSpec((B, cout), lambda b: (0, 0)),
        out_shape=jax.ShapeDtypeStruct((B, cout), jnp.float32),
    )(h, w, sc)


def _head_body(v_ref, w2_ref, gb2_ref, w3_ref, gb3_ref, w4_ref, b4_ref,
               cls_ref, xv_ref):
    def bn_batch(h, gb):
        m = jnp.mean(h, axis=0, keepdims=True)
        var = jnp.mean(h * h, axis=0, keepdims=True) - m * m
        return (h - m) / jnp.sqrt(var + EPS) * gb[0:1, :] + gb[1:2, :]

    h = _lrelu(bn_batch(_nt_bf16(v_ref[...], w2_ref[...]), gb2_ref))
    xv = _lrelu(bn_batch(_nt_bf16(h, w3_ref[...]) + gb3_ref[2:3, :], gb3_ref))
    xv_ref[...] = xv
    cls_ref[...] = _nt_bf16(xv, w4_ref[...]) + b4_ref[0:1, :]


def _head(vector, w2, g2, b2, w3, b3lin, g3, b3, w4, b4):
    B = vector.shape[0]
    gb2 = jnp.concatenate([g2[None], b2[None]], axis=0)
    gb3 = jnp.concatenate([g3[None], b3[None], b3lin[None]], axis=0)
    full = lambda shape: pl.BlockSpec(shape, lambda: tuple(0 for _ in shape))
    return pl.pallas_call(
        _head_body,
        in_specs=[full(vector.shape), full(w2.shape), full(gb2.shape),
                  full(w3.shape), full(gb3.shape), full(w4.shape),
                  full((1, b4.shape[0]))],
        out_specs=[full((B, w4.shape[0])), full((B, w3.shape[0]))],
        out_shape=[jax.ShapeDtypeStruct((B, w4.shape[0]), jnp.float32),
                   jax.ShapeDtypeStruct((B, w3.shape[0]), jnp.float32)],
    )(vector, w2, gb2, w3, gb3, w4, b4[None, :])


def _xv_row(xv_ref, wv_ref, b):
    crows = _nt_bf16(xv_ref[...], wv_ref[...])
    rio = lax.broadcasted_iota(jnp.int32, crows.shape, 0)
    return jnp.sum(jnp.where(rio == b, crows, 0.0), axis=0, keepdims=True)


def _seg_stats_body(h1_ref, h2_ref, xv_ref, wa_ref, wb_ref, wv_ref, out_ref):
    @pl.when(pl.program_id(0) == 0)
    def _():
        out_ref[...] = jnp.zeros_like(out_ref)

    crow = _xv_row(xv_ref, wv_ref, pl.program_id(0))
    pre = _nt_bf16(h1_ref[0], wa_ref[...]) + _nt_bf16(h2_ref[0], wb_ref[...]) + crow
    s = jnp.sum(pre, axis=0, keepdims=True)
    ss = jnp.sum(pre * pre, axis=0, keepdims=True)
    _acc_rows(out_ref, s, ss)


def _seg_apply_body(h1_ref, h2_ref, xv_ref, wa_ref, wb_ref, wv_ref, sc_ref,
                    out_ref):
    crow = _xv_row(xv_ref, wv_ref, pl.program_id(0))
    pre = _nt_bf16(h1_ref[0], wa_ref[...]) + _nt_bf16(h2_ref[0], wb_ref[...]) + crow
    out_ref[0] = _lrelu(pre * sc_ref[0:1, :] + sc_ref[1:2, :])


def _seg_call(body, h1, h2, xv, wa, wb, wv, sc=None):
    B, N, ca = h1.shape
    cout = wa.shape[0]
    in_specs = [
        pl.BlockSpec((1, N, ca), lambda b: (b, 0, 0)),
        pl.BlockSpec((1, N, h2.shape[2]), lambda b: (b, 0, 0)),
        pl.BlockSpec(xv.shape, lambda b: (0, 0)),
        pl.BlockSpec(wa.shape, lambda b: (0, 0)),
        pl.BlockSpec(wb.shape, lambda b: (0, 0)),
        pl.BlockSpec(wv.shape, lambda b: (0, 0)),
    ]
    args = [h1, h2, xv, wa, wb, wv]
    if sc is None:
        out_spec = pl.BlockSpec((8, cout), lambda b: (0, 0))
        out_shape = jax.ShapeDtypeStruct((8, cout), jnp.float32)
    else:
        in_specs.append(pl.BlockSpec((8, cout), lambda b: (0, 0)))
        args.append(sc)
        out_spec = pl.BlockSpec((1, N, cout), lambda b: (b, 0, 0))
        out_shape = jax.ShapeDtypeStruct((B, N, cout), jnp.float32)
    return pl.pallas_call(
        body,
        grid=(B,),
        in_specs=in_specs,
        out_specs=out_spec,
        out_shape=out_shape,
    )(*args)


def kernel(x, params):
    p = params
    B, _, N = x.shape
    xt = jnp.transpose(x, (0, 2, 1))
    xt8 = jnp.pad(xt, ((0, 0), (0, 0), (0, 5)))
    xt64 = jnp.pad(xt, ((0, 0), (0, 0), (0, 61)))

    w1 = p['ec0_w1']
    w1p0 = jnp.pad(jnp.concatenate([
        jnp.pad(w1[:, :3], ((0, 0), (0, 5))),
        jnp.pad(w1[:, 3:], ((0, 0), (0, 5)))], axis=1),
        ((0, 0), (0, 112)))
    gidx0 = _knn_topk(xt64).reshape(-1)
    x0cat = _edge_conv(xt8, gidx0, w1p0, p['ec0_w2'],
                       p['ec0_g1'], p['ec0_b1'], p['ec0_g2'], p['ec0_b2'],
                       dup_out=True)

    pn0_wp = jnp.pad(p['pn0_w'], ((0, 0), (0, 64)))
    st = _lin_stats(x0cat, pn0_wp)
    sc = _stats_to_scale(st, p['pn0_g'], p['pn0_b'], B * N)
    x_t0 = _lin_max(x0cat, pn0_wp, sc)

    gidx1 = _knn_topk(x0cat).reshape(-1)
    x1 = _edge_conv(x0cat, gidx1, p['ec1_w1'], p['ec1_w2'],
                    p['ec1_g1'], p['ec1_b1'], p['ec1_g2'], p['ec1_b2'])

    st = _lin_stats(x1, p['pn1_w'])
    sc = _stats_to_scale(st, p['pn1_g'], p['pn1_b'], B * N)
    x_t1 = _lin_max(x1, p['pn1_w'], sc)

    vector = jnp.concatenate([x_t0, x_t1], axis=1)
    logits_cls, x_vec = _head(vector, p['lin2_w'], p['bn2_g'], p['bn2_b'],
                              p['lin3_w'], p['lin3_b'], p['bn3_g'], p['bn3_b'],
                              p['lin4_w'], p['lin4_b'])

    w5 = p['pn5_w']
    w5v, w5a = w5[:, :256], w5[:, 256:]
    zb5 = jnp.zeros((128, 128), jnp.float32)
    st = _seg_call(_seg_stats_body, x1, x0cat, x_vec, w5a, zb5, w5v)
    sc5 = _stats_to_scale(st, p['pn5_g'], p['pn5_b'], B * N)
    h5 = _seg_call(_seg_apply_body, x1, x0cat, x_vec, w5a, zb5, w5v, sc5)

    w6 = p['pn6_w']
    w6a = w6[:, :128]
    w6bp = jnp.pad(w6[:, 128:], ((0, 0), (0, 64)))
    zv6 = jnp.zeros((128, x_vec.shape[1]), jnp.float32)
    st = _seg_call(_seg_stats_body, h5, x0cat, x_vec, w6a, w6bp, zv6)
    sc6 = _stats_to_scale(st, p['pn6_g'], p['pn6_b'], B * N)
    h6 = _seg_call(_seg_apply_body, h5, x0cat, x_vec, w6a, w6bp, zv6, sc6)

    w7 = jnp.pad(p['conv7_w'], ((0, 6), (0, 0)))
    seg8 = _bmm(h6, w7)
    logits_seg = jnp.transpose(seg8[:, :, :2], (0, 2, 1))

    return logits_cls, logits_seg, x, x


# speedup vs baseline: 1.0422x; 1.0422x over previous
import functools

import jax
import jax.numpy as jnp
from jax import lax
from jax.experimental import pallas as pl
from jax.experimental.pallas import tpu as pltpu
from jax.experimental.pallas import tpu_sc as plsc

KNB = 20
NPTS = 2048
NBATCH = 8
RBLK = 256
NPB = 128
NBLK = NPTS // NPB
EPS = 1e-5


def _lrelu(t):
    return jnp.where(t > 0, t, 0.2 * t)


def _nt(a, b):
    return lax.dot_general(a, b, (((1,), (1,)), ((), ())),
                           preferred_element_type=jnp.float32,
                           precision=lax.Precision.HIGHEST)


def _cast16(t):
    return t.astype(jnp.bfloat16)


def _nt_bf16(a, b):
    return lax.dot_general(a.astype(jnp.bfloat16), b.astype(jnp.bfloat16),
                           (((1,), (1,)), ((), ())),
                           preferred_element_type=jnp.float32)


def _knn_body(xb_ref, xa_ref, out_ref, pair_ref):
    b = pl.program_id(0)
    xb = xb_ref[0]
    xa = xa_ref[0]
    xxa = _nt(jnp.ones((1, xa.shape[1]), jnp.float32), xa * xa)
    xxb = jnp.sum(xb * xb, axis=1, keepdims=True)
    g = _nt_bf16(xb, xa)
    pair_ref[...] = 2.0 * g - xxb - xxa
    iota = lax.broadcasted_iota(jnp.int32, (RBLK, NPTS), 1)
    kio = lax.broadcasted_iota(jnp.int32, (RBLK, KNB), 1)

    def body(i, _):
        p = pair_ref[...]
        vmax = jnp.max(p, axis=1, keepdims=True)
        cand = jnp.where(p == vmax, iota, NPTS)
        amin = jnp.min(cand, axis=1, keepdims=True)
        out_ref[0] = jnp.where(kio == i,
                               jnp.broadcast_to(amin + b * NPTS, (RBLK, KNB)),
                               out_ref[0])
        pair_ref[...] = jnp.where(iota == amin, -jnp.inf, p)
        return 0

    lax.fori_loop(0, KNB, body, 0, unroll=2)


def _knn_topk(xt):
    B, N, C = xt.shape
    grid = (B, N // RBLK)
    return pl.pallas_call(
        _knn_body,
        grid=grid,
        in_specs=[
            pl.BlockSpec((1, RBLK, C), lambda b, r: (b, r, 0)),
            pl.BlockSpec((1, N, C), lambda b, r: (b, 0, 0)),
        ],
        out_specs=pl.BlockSpec((1, RBLK, KNB), lambda b, r: (b, r, 0)),
        out_shape=jax.ShapeDtypeStruct((B, N, KNB), jnp.int32),
        scratch_shapes=[pltpu.VMEM((RBLK, NPTS), jnp.float32)],
    )(xt, xt)


def _bmm_body(h_ref, w_ref, o_ref):
    o_ref[0] = _nt_bf16(h_ref[0], w_ref[...])


def _bmm(h, w):
    B, N, Cin = h.shape
    Cout = w.shape[0]
    return pl.pallas_call(
        _bmm_body,
        grid=(B,),
        in_specs=[
            pl.BlockSpec((1, N, Cin), lambda b: (b, 0, 0)),
            pl.BlockSpec((Cout, Cin), lambda b: (0, 0)),
        ],
        out_specs=pl.BlockSpec((1, N, Cout), lambda b: (b, 0, 0)),
        out_shape=jax.ShapeDtypeStruct((B, N, Cout), jnp.float32),
    )(h, w)


def _sc_gather(table, idx):
    R, C = table.shape
    (M,) = idx.shape
    info = plsc.get_sparse_core_info()
    nw = info.num_cores * info.num_subcores
    m_per_w = M // nw
    ch = 256
    n_ch = m_per_w // ch
    assert m_per_w % ch == 0 and n_ch % 2 == 0
    mesh = plsc.VectorSubcoreMesh(core_axis_name="c", subcore_axis_name="s")

    @functools.partial(
        pl.kernel, mesh=mesh,
        out_type=jax.ShapeDtypeStruct((M, C), jnp.float32),
        scratch_types=[
            pltpu.VMEM((ch,), jnp.int32),
            pltpu.VMEM((ch,), jnp.int32),
            pltpu.VMEM((ch, C), jnp.float32),
            pltpu.VMEM((ch, C), jnp.float32),
            pltpu.SemaphoreType.DMA,
            pltpu.SemaphoreType.DMA,
        ],
    )
    def k(table_hbm, idx_hbm, out_hbm, idx0, idx1, rows0, rows1, s0, s1):
        wid = lax.axis_index("s") * info.num_cores + lax.axis_index("c")
        base = wid * m_per_w

        def issue(j, idx_v, rows_v, sem):
            off = base + j * ch
            pltpu.sync_copy(idx_hbm.at[pl.ds(off, ch)], idx_v)
            pltpu.async_copy(table_hbm.at[idx_v], rows_v, sem)

        def drain(j, idx_v, rows_v, sem):
            pltpu.make_async_copy(table_hbm.at[idx_v], rows_v, sem).wait()
            pltpu.sync_copy(rows_v, out_hbm.at[pl.ds(base + j * ch, ch)])

        issue(0, idx0, rows0, s0)

        def step2(jj, _):
            j = jj * 2
            issue(j + 1, idx1, rows1, s1)
            drain(j, idx0, rows0, s0)

            @pl.when(jj + 1 < n_ch // 2)
            def _():
                issue(j + 2, idx0, rows0, s0)

            drain(j + 1, idx1, rows1, s1)
            return 0

        lax.fori_loop(0, n_ch // 2, step2, 0)

    return k(table, idx)


def _acc_rows(out_ref, s, ss):
    rio = lax.broadcasted_iota(jnp.int32, out_ref.shape, 0)
    upd = jnp.where(rio == 0, jnp.broadcast_to(s, out_ref.shape), 0.0)
    upd = upd + jnp.where(rio == 1, jnp.broadcast_to(ss, out_ref.shape), 0.0)
    out_ref[...] += upd


def _h1pre(nbr_ref, z_ref, w1_ref):
    cv = z_ref.shape[-1]
    nb = nbr_ref[0, 0][:, :cv]
    ctr = jnp.broadcast_to(z_ref[0, 0][:, None, :], (NPB, KNB, cv))
    ctr = ctr.reshape(NPB * KNB, cv)
    d16 = _cast16(nb - ctr)
    c16 = _cast16(ctr)
    parts = [d16, c16]
    if 2 * cv < 128:
        parts.append(jnp.zeros((NPB * KNB, 128 - 2 * cv), d16.dtype))
    feat = jnp.concatenate(parts, axis=1)
    return _nt_bf16(feat, w1_ref[...])


def _ec_stats1_body(nbr_ref, z_ref, w1_ref, out_ref):
    @pl.when(jnp.logical_and(pl.program_id(0) == 0, pl.program_id(1) == 0))
    def _():
        out_ref[...] = jnp.zeros_like(out_ref)

    hh = _h1pre(nbr_ref, z_ref, w1_ref)
    s = jnp.sum(hh, axis=0, keepdims=True)
    ss = jnp.sum(hh * hh, axis=0, keepdims=True)
    _acc_rows(out_ref, s, ss)


def _ec_stats2_body(nbr_ref, z_ref, w1_ref, sc1_ref, w2_ref, out_ref):
    @pl.when(jnp.logical_and(pl.program_id(0) == 0, pl.program_id(1) == 0))
    def _():
        out_ref[...] = jnp.zeros_like(out_ref)

    hh = _h1pre(nbr_ref, z_ref, w1_ref)
    h1 = _lrelu(hh * sc1_ref[0:1, :] + sc1_ref[1:2, :])
    g = _nt_bf16(h1, w2_ref[...])
    s = jnp.sum(g, axis=0, keepdims=True)
    ss = jnp.sum(g * g, axis=0, keepdims=True)
    _acc_rows(out_ref, s, ss)


def _ec_apply_body(nbr_ref, z_ref, w1_ref, sc1_ref, w2_ref, sc2_ref, out_ref,
                   *, dup_out):
    c2 = w2_ref.shape[0]
    hh = _h1pre(nbr_ref, z_ref, w1_ref)
    h1 = _lrelu(hh * sc1_ref[0:1, :] + sc1_ref[1:2, :])
    g = _nt_bf16(h1, w2_ref[...])
    t = _lrelu(g * sc2_ref[0:1, :] + sc2_ref[1:2, :])
    m = jnp.max(t.reshape(NPB, KNB, c2), axis=1)
    if dup_out:
        m = jnp.concatenate([m, m], axis=1)
    out_ref[0, 0] = m


def _ec_grid_call(body, nbr4, z4, extras, out_shape, out_spec):
    B = z4.shape[0]
    c1 = z4.shape[-1]
    cg = nbr4.shape[-1]
    in_specs = [
        pl.BlockSpec((1, 1, NPB * KNB, cg), lambda b, r: (b, r, 0, 0)),
        pl.BlockSpec((1, 1, NPB, c1), lambda b, r: (b, r, 0, 0)),
    ]
    args = [nbr4, z4]
    for e in extras:
        in_specs.append(
            pl.BlockSpec(e.shape, (lambda nd: lambda b, r: (0,) * nd)(e.ndim)))
        args.append(e)
    return pl.pallas_call(
        body,
        grid=(B, NBLK),
        in_specs=in_specs,
        out_specs=out_spec,
        out_shape=out_shape,
    )(*args)


def _stats_to_scale(st, g, b, cnt):
    mean = st[0] / cnt
    var = st[1] / cnt - mean * mean
    s = g / jnp.sqrt(var + EPS)
    o = b - mean * s
    pad = jnp.zeros((6, s.shape[0]), jnp.float32)
    return jnp.concatenate([s[None, :], o[None, :], pad], axis=0)


def _edge_conv(xr, gidx_flat, w1p, w2, g1, b1, g2, b2, dup_out=False):
    B, N, cv = xr.shape
    c1 = w1p.shape[0]
    c2 = w2.shape[0]
    xp = xr if cv == 128 else jnp.pad(xr, ((0, 0), (0, 0), (0, 128 - cv)))
    nbr = _sc_gather(xp.reshape(B * N, 128), gidx_flat)
    nbr4 = nbr.reshape(B, NBLK, NPB * KNB, 128)
    z4 = xr.reshape(B, NBLK, NPB, cv)
    cnt = B * N * KNB
    st1 = _ec_grid_call(
        _ec_stats1_body, nbr4, z4, [w1p],
        jax.ShapeDtypeStruct((8, c1), jnp.float32),
        pl.BlockSpec((8, c1), lambda b, r: (0, 0)))
    sc1 = _stats_to_scale(st1, g1, b1, cnt)
    st2 = _ec_grid_call(
        _ec_stats2_body, nbr4, z4, [w1p, sc1, w2],
        jax.ShapeDtypeStruct((8, c2), jnp.float32),
        pl.BlockSpec((8, c2), lambda b, r: (0, 0)))
    sc2 = _stats_to_scale(st2, g2, b2, cnt)
    c_out = 2 * c2 if dup_out else c2
    out = _ec_grid_call(
        functools.partial(_ec_apply_body, dup_out=dup_out), nbr4, z4,
        [w1p, sc1, w2, sc2],
        jax.ShapeDtypeStruct((B, NBLK, NPB, c_out), jnp.float32),
        pl.BlockSpec((1, 1, NPB, c_out), lambda b, r: (b, r, 0, 0)))
    return out.reshape(B, N, c_out)


def _lin_stats_body(h_ref, w_ref, out_ref):
    @pl.when(pl.program_id(0) == 0)
    def _():
        out_ref[...] = jnp.zeros_like(out_ref)

    pre = _nt_bf16(h_ref[0], w_ref[...])
    s = jnp.sum(pre, axis=0, keepdims=True)
    ss = jnp.sum(pre * pre, axis=0, keepdims=True)
    _acc_rows(out_ref, s, ss)


def _lin_stats(h, w):
    B, N, cin = h.shape
    cout = w.shape[0]
    return pl.pallas_call(
        _lin_stats_body,
        grid=(B,),
        in_specs=[
            pl.BlockSpec((1, N, cin), lambda b: (b, 0, 0)),
            pl.BlockSpec((cout, cin), lambda b: (0, 0)),
        ],
        out_specs=pl.BlockSpec((8, cout), lambda b: (0, 0)),
        out_shape=jax.ShapeDtypeStruct((8, cout), jnp.float32),
    )(h, w)


def _lin_max_body(h_ref, w_ref, sc_ref, out_ref):
    b = pl.program_id(0)
    pre = _nt_bf16(h_ref[0], w_ref[...])
    act = _lrelu(pre * sc_ref[0:1, :] + sc_ref[1:2, :])
    row = jnp.max(act, axis=0, keepdims=True)
    rio = lax.broadcasted_iota(jnp.int32, out_ref.shape, 0)
    out_ref[...] = jnp.where(rio == b, jnp.broadcast_to(row, out_ref.shape),
                             out_ref[...])


def _lin_max(h, w, sc):
    B, N, cin = h.shape
    cout = w.shape[0]
    return pl.pallas_call(
        _lin_max_body,
        grid=(B,),
        in_specs=[
            pl.BlockSpec((1, N, cin), lambda b: (b, 0, 0)),
            pl.BlockSpec((cout, cin), lambda b: (0, 0)),
            pl.BlockSpec((8, cout), lambda b: (0, 0)),
        ],
        out_specs=pl.BlockSpec((B, cout), lambda b: (0, 0)),
        out_shape=jax.ShapeDtypeStruct((B, cout), jnp.float32),
    )(h, w, sc)


def _head_body(v_ref, w2_ref, gb2_ref, w3_ref, gb3_ref, w4_ref, b4_ref,
               cls_ref, xv_ref):
    def bn_batch(h, gb):
        m = jnp.mean(h, axis=0, keepdims=True)
        var = jnp.mean(h * h, axis=0, keepdims=True) - m * m
        return (h - m) / jnp.sqrt(var + EPS) * gb[0:1, :] + gb[1:2, :]

    h = _lrelu(bn_batch(_nt_bf16(v_ref[...], w2_ref[...]), gb2_ref))
    xv = _lrelu(bn_batch(_nt_bf16(h, w3_ref[...]) + gb3_ref[2:3, :], gb3_ref))
    xv_ref[...] = xv
    cls_ref[...] = _nt_bf16(xv, w4_ref[...]) + b4_ref[0:1, :]


def _head(vector, w2, g2, b2, w3, b3lin, g3, b3, w4, b4):
    B = vector.shape[0]
    gb2 = jnp.concatenate([g2[None], b2[None]], axis=0)
    gb3 = jnp.concatenate([g3[None], b3[None], b3lin[None]], axis=0)
    full = lambda shape: pl.BlockSpec(shape, lambda: tuple(0 for _ in shape))
    return pl.pallas_call(
        _head_body,
        in_specs=[full(vector.shape), full(w2.shape), full(gb2.shape),
                  full(w3.shape), full(gb3.shape), full(w4.shape),
                  full((1, b4.shape[0]))],
        out_specs=[full((B, w4.shape[0])), full((B, w3.shape[0]))],
        out_shape=[jax.ShapeDtypeStruct((B, w4.shape[0]), jnp.float32),
                   jax.ShapeDtypeStruct((B, w3.shape[0]), jnp.float32)],
    )(vector, w2, gb2, w3, gb3, w4, b4[None, :])


def _xv_row(xv_ref, wv_ref, b):
    crows = _nt_bf16(xv_ref[...], wv_ref[...])
    rio = lax.broadcasted_iota(jnp.int32, crows.shape, 0)
    return jnp.sum(jnp.where(rio == b, crows, 0.0), axis=0, keepdims=True)


def _seg_stats_body(h1_ref, h2_ref, xv_ref, wa_ref, wb_ref, wv_ref, out_ref):
    @pl.when(pl.program_id(0) == 0)
    def _():
        out_ref[...] = jnp.zeros_like(out_ref)

    crow = _xv_row(xv_ref, wv_ref, pl.program_id(0))
    pre = _nt_bf16(h1_ref[0], wa_ref[...]) + _nt_bf16(h2_ref[0], wb_ref[...]) + crow
    s = jnp.sum(pre, axis=0, keepdims=True)
    ss = jnp.sum(pre * pre, axis=0, keepdims=True)
    _acc_rows(out_ref, s, ss)


def _seg_apply_body(h1_ref, h2_ref, xv_ref, wa_ref, wb_ref, wv_ref, sc_ref,
                    out_ref):
    crow = _xv_row(xv_ref, wv_ref, pl.program_id(0))
    pre = _nt_bf16(h1_ref[0], wa_ref[...]) + _nt_bf16(h2_ref[0], wb_ref[...]) + crow
    out_ref[0] = _lrelu(pre * sc_ref[0:1, :] + sc_ref[1:2, :])


def _seg_call(body, h1, h2, xv, wa, wb, wv, sc=None):
    B, N, ca = h1.shape
    cout = wa.shape[0]
    in_specs = [
        pl.BlockSpec((1, N, ca), lambda b: (b, 0, 0)),
        pl.BlockSpec((1, N, h2.shape[2]), lambda b: (b, 0, 0)),
        pl.BlockSpec(xv.shape, lambda b: (0, 0)),
        pl.BlockSpec(wa.shape, lambda b: (0, 0)),
        pl.BlockSpec(wb.shape, lambda b: (0, 0)),
        pl.BlockSpec(wv.shape, lambda b: (0, 0)),
    ]
    args = [h1, h2, xv, wa, wb, wv]
    if sc is None:
        out_spec = pl.BlockSpec((8, cout), lambda b: (0, 0))
        out_shape = jax.ShapeDtypeStruct((8, cout), jnp.float32)
    else:
        in_specs.append(pl.BlockSpec((8, cout), lambda b: (0, 0)))
        args.append(sc)
        out_spec = pl.BlockSpec((1, N, cout), lambda b: (b, 0, 0))
        out_shape = jax.ShapeDtypeStruct((B, N, cout), jnp.float32)
    return pl.pallas_call(
        body,
        grid=(B,),
        in_specs=in_specs,
        out_specs=out_spec,
        out_shape=out_shape,
    )(*args)


def kernel(x, params):
    p = params
    B, _, N = x.shape
    xt = jnp.transpose(x, (0, 2, 1))
    xt8 = jnp.pad(xt, ((0, 0), (0, 0), (0, 5)))
    xt64 = jnp.pad(xt, ((0, 0), (0, 0), (0, 61)))

    w1 = p['ec0_w1']
    w1p0 = jnp.pad(jnp.concatenate([
        jnp.pad(w1[:, :3], ((0, 0), (0, 5))),
        jnp.pad(w1[:, 3:], ((0, 0), (0, 5)))], axis=1),
        ((0, 0), (0, 112)))
    gidx0 = _knn_topk(xt64).reshape(-1)
    x0cat = _edge_conv(xt8, gidx0, w1p0, p['ec0_w2'],
                       p['ec0_g1'], p['ec0_b1'], p['ec0_g2'], p['ec0_b2'],
                       dup_out=True)

    pn0_wp = jnp.pad(p['pn0_w'], ((0, 0), (0, 64)))
    st = _lin_stats(x0cat, pn0_wp)
    sc = _stats_to_scale(st, p['pn0_g'], p['pn0_b'], B * N)
    x_t0 = _lin_max(x0cat, pn0_wp, sc)

    gidx1 = _knn_topk(x0cat).reshape(-1)
    x1 = _edge_conv(x0cat, gidx1, p['ec1_w1'], p['ec1_w2'],
                    p['ec1_g1'], p['ec1_b1'], p['ec1_g2'], p['ec1_b2'])

    st = _lin_stats(x1, p['pn1_w'])
    sc = _stats_to_scale(st, p['pn1_g'], p['pn1_b'], B * N)
    x_t1 = _lin_max(x1, p['pn1_w'], sc)

    vector = jnp.concatenate([x_t0, x_t1], axis=1)
    logits_cls, x_vec = _head(vector, p['lin2_w'], p['bn2_g'], p['bn2_b'],
                              p['lin3_w'], p['lin3_b'], p['bn3_g'], p['bn3_b'],
                              p['lin4_w'], p['lin4_b'])

    w5 = p['pn5_w']
    w5v, w5a = w5[:, :256], w5[:, 256:]
    zb5 = jnp.zeros((128, 128), jnp.float32)
    st = _seg_call(_seg_stats_body, x1, x0cat, x_vec, w5a, zb5, w5v)
    sc5 = _stats_to_scale(st, p['pn5_g'], p['pn5_b'], B * N)
    h5 = _seg_call(_seg_apply_body, x1, x0cat, x_vec, w5a, zb5, w5v, sc5)

    w6 = p['pn6_w']
    w6a = w6[:, :128]
    w6bp = jnp.pad(w6[:, 128:], ((0, 0), (0, 64)))
    zv6 = jnp.zeros((128, x_vec.shape[1]), jnp.float32)
    st = _seg_call(_seg_stats_body, h5, x0cat, x_vec, w6a, w6bp, zv6)
    sc6 = _stats_to_scale(st, p['pn6_g'], p['pn6_b'], B * N)
    h6 = _seg_call(_seg_apply_body, h5, x0cat, x_vec, w6a, w6bp, zv6, sc6)

    w7 = jnp.pad(p['conv7_w'], ((0, 6), (0, 0)))
    seg8 = _bmm(h6, w7)
    logits_seg = jnp.transpose(seg8[:, :, :2], (0, 2, 1))

    return logits_cls, logits_seg, x, x


# speedup vs baseline: 1.1228x; 1.0773x over previous
import functools

import jax
import jax.numpy as jnp
from jax import lax
from jax.experimental import pallas as pl
from jax.experimental.pallas import tpu as pltpu
from jax.experimental.pallas import tpu_sc as plsc

KNB = 20
NPTS = 2048
NBATCH = 8
RBLK = 512
NPB = 128
NBLK = NPTS // NPB
EPS = 1e-5


def _lrelu(t):
    return jnp.where(t > 0, t, 0.2 * t)


def _nt(a, b):
    return lax.dot_general(a, b, (((1,), (1,)), ((), ())),
                           preferred_element_type=jnp.float32,
                           precision=lax.Precision.HIGHEST)


def _cast16(t):
    return t.astype(jnp.bfloat16)


def _nt_bf16(a, b):
    return lax.dot_general(a.astype(jnp.bfloat16), b.astype(jnp.bfloat16),
                           (((1,), (1,)), ((), ())),
                           preferred_element_type=jnp.float32)


def _knn_body(xb_ref, xa_ref, out_ref, pair_ref):
    b = pl.program_id(0)
    xb = xb_ref[0]
    xa = xa_ref[0]
    xxa = _nt(jnp.ones((1, xa.shape[1]), jnp.float32), xa * xa)
    xxb = jnp.sum(xb * xb, axis=1, keepdims=True)
    g = _nt_bf16(xb, xa)
    pair_ref[...] = 2.0 * g - xxb - xxa
    iota = lax.broadcasted_iota(jnp.int32, (RBLK, NPTS), 1)
    kio = lax.broadcasted_iota(jnp.int32, (RBLK, KNB), 1)

    def body(i, _):
        p = pair_ref[...]
        vmax = jnp.max(p, axis=1, keepdims=True)
        cand = jnp.where(p == vmax, iota, NPTS)
        amin = jnp.min(cand, axis=1, keepdims=True)
        out_ref[0] = jnp.where(kio == i,
                               jnp.broadcast_to(amin + b * NPTS, (RBLK, KNB)),
                               out_ref[0])
        pair_ref[...] = jnp.where(iota == amin, -jnp.inf, p)
        return 0

    lax.fori_loop(0, KNB, body, 0, unroll=2)


def _knn_topk(xt):
    B, N, C = xt.shape
    grid = (B, N // RBLK)
    return pl.pallas_call(
        _knn_body,
        grid=grid,
        in_specs=[
            pl.BlockSpec((1, RBLK, C), lambda b, r: (b, r, 0)),
            pl.BlockSpec((1, N, C), lambda b, r: (b, 0, 0)),
        ],
        out_specs=pl.BlockSpec((1, RBLK, KNB), lambda b, r: (b, r, 0)),
        out_shape=jax.ShapeDtypeStruct((B, N, KNB), jnp.int32),
        scratch_shapes=[pltpu.VMEM((RBLK, NPTS), jnp.float32)],
    )(xt, xt)


def _bmm_body(h_ref, w_ref, o_ref):
    o_ref[0] = _nt_bf16(h_ref[0], w_ref[...])


def _bmm(h, w):
    B, N, Cin = h.shape
    Cout = w.shape[0]
    return pl.pallas_call(
        _bmm_body,
        grid=(B,),
        in_specs=[
            pl.BlockSpec((1, N, Cin), lambda b: (b, 0, 0)),
            pl.BlockSpec((Cout, Cin), lambda b: (0, 0)),
        ],
        out_specs=pl.BlockSpec((1, N, Cout), lambda b: (b, 0, 0)),
        out_shape=jax.ShapeDtypeStruct((B, N, Cout), jnp.float32),
    )(h, w)


def _sc_gather(table, idx):
    R, C = table.shape
    (M,) = idx.shape
    info = plsc.get_sparse_core_info()
    nw = info.num_cores * info.num_subcores
    m_per_w = M // nw
    ch = 256
    n_ch = m_per_w // ch
    assert m_per_w % ch == 0 and n_ch % 2 == 0
    mesh = plsc.VectorSubcoreMesh(core_axis_name="c", subcore_axis_name="s")

    @functools.partial(
        pl.kernel, mesh=mesh,
        out_type=jax.ShapeDtypeStruct((M, C), jnp.float32),
        scratch_types=[
            pltpu.VMEM((ch,), jnp.int32),
            pltpu.VMEM((ch,), jnp.int32),
            pltpu.VMEM((ch, C), jnp.float32),
            pltpu.VMEM((ch, C), jnp.float32),
            pltpu.SemaphoreType.DMA,
            pltpu.SemaphoreType.DMA,
        ],
    )
    def k(table_hbm, idx_hbm, out_hbm, idx0, idx1, rows0, rows1, s0, s1):
        wid = lax.axis_index("s") * info.num_cores + lax.axis_index("c")
        base = wid * m_per_w

        def issue(j, idx_v, rows_v, sem):
            off = base + j * ch
            pltpu.sync_copy(idx_hbm.at[pl.ds(off, ch)], idx_v)
            pltpu.async_copy(table_hbm.at[idx_v], rows_v, sem)

        def drain(j, idx_v, rows_v, sem):
            pltpu.make_async_copy(table_hbm.at[idx_v], rows_v, sem).wait()
            pltpu.sync_copy(rows_v, out_hbm.at[pl.ds(base + j * ch, ch)])

        issue(0, idx0, rows0, s0)

        def step2(jj, _):
            j = jj * 2
            issue(j + 1, idx1, rows1, s1)
            drain(j, idx0, rows0, s0)

            @pl.when(jj + 1 < n_ch // 2)
            def _():
                issue(j + 2, idx0, rows0, s0)

            drain(j + 1, idx1, rows1, s1)
            return 0

        lax.fori_loop(0, n_ch // 2, step2, 0)

    return k(table, idx)


def _acc_rows(out_ref, s, ss):
    rio = lax.broadcasted_iota(jnp.int32, out_ref.shape, 0)
    upd = jnp.where(rio == 0, jnp.broadcast_to(s, out_ref.shape), 0.0)
    upd = upd + jnp.where(rio == 1, jnp.broadcast_to(ss, out_ref.shape), 0.0)
    out_ref[...] += upd


def _h1pre(nbr_ref, z_ref, w1_ref):
    cv = z_ref.shape[-1]
    nb = nbr_ref[0, 0][:, :cv]
    ctr = jnp.broadcast_to(z_ref[0, 0][:, None, :], (NPB, KNB, cv))
    ctr = ctr.reshape(NPB * KNB, cv)
    d16 = _cast16(nb - ctr)
    c16 = _cast16(ctr)
    parts = [d16, c16]
    if 2 * cv < 128:
        parts.append(jnp.zeros((NPB * KNB, 128 - 2 * cv), d16.dtype))
    feat = jnp.concatenate(parts, axis=1)
    return _nt_bf16(feat, w1_ref[...])


def _ec_stats1_body(nbr_ref, z_ref, w1_ref, out_ref):
    @pl.when(jnp.logical_and(pl.program_id(0) == 0, pl.program_id(1) == 0))
    def _():
        out_ref[...] = jnp.zeros_like(out_ref)

    hh = _h1pre(nbr_ref, z_ref, w1_ref)
    s = jnp.sum(hh, axis=0, keepdims=True)
    ss = jnp.sum(hh * hh, axis=0, keepdims=True)
    _acc_rows(out_ref, s, ss)


def _ec_stats2_body(nbr_ref, z_ref, w1_ref, sc1_ref, w2_ref, out_ref):
    @pl.when(jnp.logical_and(pl.program_id(0) == 0, pl.program_id(1) == 0))
    def _():
        out_ref[...] = jnp.zeros_like(out_ref)

    hh = _h1pre(nbr_ref, z_ref, w1_ref)
    h1 = _lrelu(hh * sc1_ref[0:1, :] + sc1_ref[1:2, :])
    g = _nt_bf16(h1, w2_ref[...])
    s = jnp.sum(g, axis=0, keepdims=True)
    ss = jnp.sum(g * g, axis=0, keepdims=True)
    _acc_rows(out_ref, s, ss)


def _ec_apply_body(nbr_ref, z_ref, w1_ref, sc1_ref, w2_ref, sc2_ref, out_ref,
                   *, dup_out):
    c2 = w2_ref.shape[0]
    hh = _h1pre(nbr_ref, z_ref, w1_ref)
    h1 = _lrelu(hh * sc1_ref[0:1, :] + sc1_ref[1:2, :])
    g = _nt_bf16(h1, w2_ref[...])
    t = _lrelu(g * sc2_ref[0:1, :] + sc2_ref[1:2, :])
    m = jnp.max(t.reshape(NPB, KNB, c2), axis=1)
    if dup_out:
        m = jnp.concatenate([m, m], axis=1)
    out_ref[0, 0] = m


def _ec_grid_call(body, nbr4, z4, extras, out_shape, out_spec):
    B = z4.shape[0]
    c1 = z4.shape[-1]
    cg = nbr4.shape[-1]
    in_specs = [
        pl.BlockSpec((1, 1, NPB * KNB, cg), lambda b, r: (b, r, 0, 0)),
        pl.BlockSpec((1, 1, NPB, c1), lambda b, r: (b, r, 0, 0)),
    ]
    args = [nbr4, z4]
    for e in extras:
        in_specs.append(
            pl.BlockSpec(e.shape, (lambda nd: lambda b, r: (0,) * nd)(e.ndim)))
        args.append(e)
    return pl.pallas_call(
        body,
        grid=(B, NBLK),
        in_specs=in_specs,
        out_specs=out_spec,
        out_shape=out_shape,
    )(*args)


def _stats_to_scale(st, g, b, cnt):
    mean = st[0] / cnt
    var = st[1] / cnt - mean * mean
    s = g / jnp.sqrt(var + EPS)
    o = b - mean * s
    pad = jnp.zeros((6, s.shape[0]), jnp.float32)
    return jnp.concatenate([s[None, :], o[None, :], pad], axis=0)


def _edge_conv(xr, gidx_flat, w1p, w2, g1, b1, g2, b2, dup_out=False):
    B, N, cv = xr.shape
    c1 = w1p.shape[0]
    c2 = w2.shape[0]
    xp = xr if cv == 128 else jnp.pad(xr, ((0, 0), (0, 0), (0, 128 - cv)))
    nbr = _sc_gather(xp.reshape(B * N, 128), gidx_flat)
    nbr4 = nbr.reshape(B, NBLK, NPB * KNB, 128)
    z4 = xr.reshape(B, NBLK, NPB, cv)
    cnt = B * N * KNB
    st1 = _ec_grid_call(
        _ec_stats1_body, nbr4, z4, [w1p],
        jax.ShapeDtypeStruct((8, c1), jnp.float32),
        pl.BlockSpec((8, c1), lambda b, r: (0, 0)))
    sc1 = _stats_to_scale(st1, g1, b1, cnt)
    st2 = _ec_grid_call(
        _ec_stats2_body, nbr4, z4, [w1p, sc1, w2],
        jax.ShapeDtypeStruct((8, c2), jnp.float32),
        pl.BlockSpec((8, c2), lambda b, r: (0, 0)))
    sc2 = _stats_to_scale(st2, g2, b2, cnt)
    c_out = 2 * c2 if dup_out else c2
    out = _ec_grid_call(
        functools.partial(_ec_apply_body, dup_out=dup_out), nbr4, z4,
        [w1p, sc1, w2, sc2],
        jax.ShapeDtypeStruct((B, NBLK, NPB, c_out), jnp.float32),
        pl.BlockSpec((1, 1, NPB, c_out), lambda b, r: (b, r, 0, 0)))
    return out.reshape(B, N, c_out)


def _lin_stats_body(h_ref, w_ref, out_ref):
    @pl.when(pl.program_id(0) == 0)
    def _():
        out_ref[...] = jnp.zeros_like(out_ref)

    pre = _nt_bf16(h_ref[0], w_ref[...])
    s = jnp.sum(pre, axis=0, keepdims=True)
    ss = jnp.sum(pre * pre, axis=0, keepdims=True)
    _acc_rows(out_ref, s, ss)


def _lin_stats(h, w):
    B, N, cin = h.shape
    cout = w.shape[0]
    return pl.pallas_call(
        _lin_stats_body,
        grid=(B,),
        in_specs=[
            pl.BlockSpec((1, N, cin), lambda b: (b, 0, 0)),
            pl.BlockSpec((cout, cin), lambda b: (0, 0)),
        ],
        out_specs=pl.BlockSpec((8, cout), lambda b: (0, 0)),
        out_shape=jax.ShapeDtypeStruct((8, cout), jnp.float32),
    )(h, w)


def _lin_max_body(h_ref, w_ref, sc_ref, out_ref):
    b = pl.program_id(0)
    pre = _nt_bf16(h_ref[0], w_ref[...])
    act = _lrelu(pre * sc_ref[0:1, :] + sc_ref[1:2, :])
    row = jnp.max(act, axis=0, keepdims=True)
    rio = lax.broadcasted_iota(jnp.int32, out_ref.shape, 0)
    out_ref[...] = jnp.where(rio == b, jnp.broadcast_to(row, out_ref.shape),
                             out_ref[...])


def _lin_max(h, w, sc):
    B, N, cin = h.shape
    cout = w.shape[0]
    return pl.pallas_call(
        _lin_max_body,
        grid=(B,),
        in_specs=[
            pl.BlockSpec((1, N, cin), lambda b: (b, 0, 0)),
            pl.BlockSpec((cout, cin), lambda b: (0, 0)),
            pl.BlockSpec((8, cout), lambda b: (0, 0)),
        ],
        out_specs=pl.BlockSpec((B, cout), lambda b: (0, 0)),
        out_shape=jax.ShapeDtypeStruct((B, cout), jnp.float32),
    )(h, w, sc)


def _head_body(v_ref, w2_ref, gb2_ref, w3_ref, gb3_ref, w4_ref, b4_ref,
               cls_ref, xv_ref):
    def bn_batch(h, gb):
        m = jnp.mean(h, axis=0, keepdims=True)
        var = jnp.mean(h * h, axis=0, keepdims=True) - m * m
        return (h - m) / jnp.sqrt(var + EPS) * gb[0:1, :] + gb[1:2, :]

    h = _lrelu(bn_batch(_nt_bf16(v_ref[...], w2_ref[...]), gb2_ref))
    xv = _lrelu(bn_batch(_nt_bf16(h, w3_ref[...]) + gb3_ref[2:3, :], gb3_ref))
    xv_ref[...] = xv
    cls_ref[...] = _nt_bf16(xv, w4_ref[...]) + b4_ref[0:1, :]


def _head(vector, w2, g2, b2, w3, b3lin, g3, b3, w4, b4):
    B = vector.shape[0]
    gb2 = jnp.concatenate([g2[None], b2[None]], axis=0)
    gb3 = jnp.concatenate([g3[None], b3[None], b3lin[None]], axis=0)
    full = lambda shape: pl.BlockSpec(shape, lambda: tuple(0 for _ in shape))
    return pl.pallas_call(
        _head_body,
        in_specs=[full(vector.shape), full(w2.shape), full(gb2.shape),
                  full(w3.shape), full(gb3.shape), full(w4.shape),
                  full((1, b4.shape[0]))],
        out_specs=[full((B, w4.shape[0])), full((B, w3.shape[0]))],
        out_shape=[jax.ShapeDtypeStruct((B, w4.shape[0]), jnp.float32),
                   jax.ShapeDtypeStruct((B, w3.shape[0]), jnp.float32)],
    )(vector, w2, gb2, w3, gb3, w4, b4[None, :])


def _xv_row(xv_ref, wv_ref, b):
    crows = _nt_bf16(xv_ref[...], wv_ref[...])
    rio = lax.broadcasted_iota(jnp.int32, crows.shape, 0)
    return jnp.sum(jnp.where(rio == b, crows, 0.0), axis=0, keepdims=True)


def _seg_stats_body(h1_ref, h2_ref, xv_ref, wa_ref, wb_ref, wv_ref, out_ref):
    @pl.when(pl.program_id(0) == 0)
    def _():
        out_ref[...] = jnp.zeros_like(out_ref)

    crow = _xv_row(xv_ref, wv_ref, pl.program_id(0))
    pre = _nt_bf16(h1_ref[0], wa_ref[...]) + _nt_bf16(h2_ref[0], wb_ref[...]) + crow
    s = jnp.sum(pre, axis=0, keepdims=True)
    ss = jnp.sum(pre * pre, axis=0, keepdims=True)
    _acc_rows(out_ref, s, ss)


def _seg_apply_body(h1_ref, h2_ref, xv_ref, wa_ref, wb_ref, wv_ref, sc_ref,
                    out_ref):
    crow = _xv_row(xv_ref, wv_ref, pl.program_id(0))
    pre = _nt_bf16(h1_ref[0], wa_ref[...]) + _nt_bf16(h2_ref[0], wb_ref[...]) + crow
    out_ref[0] = _lrelu(pre * sc_ref[0:1, :] + sc_ref[1:2, :])


def _seg_call(body, h1, h2, xv, wa, wb, wv, sc=None):
    B, N, ca = h1.shape
    cout = wa.shape[0]
    in_specs = [
        pl.BlockSpec((1, N, ca), lambda b: (b, 0, 0)),
        pl.BlockSpec((1, N, h2.shape[2]), lambda b: (b, 0, 0)),
        pl.BlockSpec(xv.shape, lambda b: (0, 0)),
        pl.BlockSpec(wa.shape, lambda b: (0, 0)),
        pl.BlockSpec(wb.shape, lambda b: (0, 0)),
        pl.BlockSpec(wv.shape, lambda b: (0, 0)),
    ]
    args = [h1, h2, xv, wa, wb, wv]
    if sc is None:
        out_spec = pl.BlockSpec((8, cout), lambda b: (0, 0))
        out_shape = jax.ShapeDtypeStruct((8, cout), jnp.float32)
    else:
        in_specs.append(pl.BlockSpec((8, cout), lambda b: (0, 0)))
        args.append(sc)
        out_spec = pl.BlockSpec((1, N, cout), lambda b: (b, 0, 0))
        out_shape = jax.ShapeDtypeStruct((B, N, cout), jnp.float32)
    return pl.pallas_call(
        body,
        grid=(B,),
        in_specs=in_specs,
        out_specs=out_spec,
        out_shape=out_shape,
    )(*args)


def kernel(x, params):
    p = params
    B, _, N = x.shape
    xt = jnp.transpose(x, (0, 2, 1))
    xt8 = jnp.pad(xt, ((0, 0), (0, 0), (0, 5)))
    xt64 = jnp.pad(xt, ((0, 0), (0, 0), (0, 61)))

    w1 = p['ec0_w1']
    w1p0 = jnp.pad(jnp.concatenate([
        jnp.pad(w1[:, :3], ((0, 0), (0, 5))),
        jnp.pad(w1[:, 3:], ((0, 0), (0, 5)))], axis=1),
        ((0, 0), (0, 112)))
    gidx0 = _knn_topk(xt64).reshape(-1)
    x0cat = _edge_conv(xt8, gidx0, w1p0, p['ec0_w2'],
                       p['ec0_g1'], p['ec0_b1'], p['ec0_g2'], p['ec0_b2'],
                       dup_out=True)

    pn0_wp = jnp.pad(p['pn0_w'], ((0, 0), (0, 64)))
    st = _lin_stats(x0cat, pn0_wp)
    sc = _stats_to_scale(st, p['pn0_g'], p['pn0_b'], B * N)
    x_t0 = _lin_max(x0cat, pn0_wp, sc)

    gidx1 = _knn_topk(x0cat).reshape(-1)
    x1 = _edge_conv(x0cat, gidx1, p['ec1_w1'], p['ec1_w2'],
                    p['ec1_g1'], p['ec1_b1'], p['ec1_g2'], p['ec1_b2'])

    st = _lin_stats(x1, p['pn1_w'])
    sc = _stats_to_scale(st, p['pn1_g'], p['pn1_b'], B * N)
    x_t1 = _lin_max(x1, p['pn1_w'], sc)

    vector = jnp.concatenate([x_t0, x_t1], axis=1)
    logits_cls, x_vec = _head(vector, p['lin2_w'], p['bn2_g'], p['bn2_b'],
                              p['lin3_w'], p['lin3_b'], p['bn3_g'], p['bn3_b'],
                              p['lin4_w'], p['lin4_b'])

    w5 = p['pn5_w']
    w5v, w5a = w5[:, :256], w5[:, 256:]
    zb5 = jnp.zeros((128, 128), jnp.float32)
    st = _seg_call(_seg_stats_body, x1, x0cat, x_vec, w5a, zb5, w5v)
    sc5 = _stats_to_scale(st, p['pn5_g'], p['pn5_b'], B * N)
    h5 = _seg_call(_seg_apply_body, x1, x0cat, x_vec, w5a, zb5, w5v, sc5)

    w6 = p['pn6_w']
    w6a = w6[:, :128]
    w6bp = jnp.pad(w6[:, 128:], ((0, 0), (0, 64)))
    zv6 = jnp.zeros((128, x_vec.shape[1]), jnp.float32)
    st = _seg_call(_seg_stats_body, h5, x0cat, x_vec, w6a, w6bp, zv6)
    sc6 = _stats_to_scale(st, p['pn6_g'], p['pn6_b'], B * N)
    h6 = _seg_call(_seg_apply_body, h5, x0cat, x_vec, w6a, w6bp, zv6, sc6)

    w7 = jnp.pad(p['conv7_w'], ((0, 6), (0, 0)))
    seg8 = _bmm(h6, w7)
    logits_seg = jnp.transpose(seg8[:, :, :2], (0, 2, 1))

    return logits_cls, logits_seg, x, x


# speedup vs baseline: 1.1487x; 1.0231x over previous
import functools

import jax
import jax.numpy as jnp
from jax import lax
from jax.experimental import pallas as pl
from jax.experimental.pallas import tpu as pltpu
from jax.experimental.pallas import tpu_sc as plsc

KNB = 20
NPTS = 2048
NBATCH = 8
RBLK = 1024
NPB = 128
NBLK = NPTS // NPB
EPS = 1e-5


def _lrelu(t):
    return jnp.where(t > 0, t, 0.2 * t)


def _nt(a, b):
    return lax.dot_general(a, b, (((1,), (1,)), ((), ())),
                           preferred_element_type=jnp.float32,
                           precision=lax.Precision.HIGHEST)


def _cast16(t):
    return t.astype(jnp.bfloat16)


def _nt_bf16(a, b):
    return lax.dot_general(a.astype(jnp.bfloat16), b.astype(jnp.bfloat16),
                           (((1,), (1,)), ((), ())),
                           preferred_element_type=jnp.float32)


def _knn_body(xb_ref, xa_ref, out_ref, pair_ref):
    b = pl.program_id(0)
    xb = xb_ref[0]
    xa = xa_ref[0]
    xxa = _nt(jnp.ones((1, xa.shape[1]), jnp.float32), xa * xa)
    xxb = jnp.sum(xb * xb, axis=1, keepdims=True)
    g = _nt_bf16(xb, xa)
    pair_ref[...] = 2.0 * g - xxb - xxa
    iota = lax.broadcasted_iota(jnp.int32, (RBLK, NPTS), 1)
    kio = lax.broadcasted_iota(jnp.int32, (RBLK, KNB), 1)

    def body(i, _):
        p = pair_ref[...]
        vmax = jnp.max(p, axis=1, keepdims=True)
        cand = jnp.where(p == vmax, iota, NPTS)
        amin = jnp.min(cand, axis=1, keepdims=True)
        out_ref[0] = jnp.where(kio == i,
                               jnp.broadcast_to(amin + b * NPTS, (RBLK, KNB)),
                               out_ref[0])
        pair_ref[...] = jnp.where(iota == amin, -jnp.inf, p)
        return 0

    lax.fori_loop(0, KNB, body, 0, unroll=2)


def _knn_topk(xt):
    B, N, C = xt.shape
    grid = (B, N // RBLK)
    return pl.pallas_call(
        _knn_body,
        grid=grid,
        in_specs=[
            pl.BlockSpec((1, RBLK, C), lambda b, r: (b, r, 0)),
            pl.BlockSpec((1, N, C), lambda b, r: (b, 0, 0)),
        ],
        out_specs=pl.BlockSpec((1, RBLK, KNB), lambda b, r: (b, r, 0)),
        out_shape=jax.ShapeDtypeStruct((B, N, KNB), jnp.int32),
        scratch_shapes=[pltpu.VMEM((RBLK, NPTS), jnp.float32)],
    )(xt, xt)


def _bmm_body(h_ref, w_ref, o_ref):
    o_ref[0] = _nt_bf16(h_ref[0], w_ref[...])


def _bmm(h, w):
    B, N, Cin = h.shape
    Cout = w.shape[0]
    return pl.pallas_call(
        _bmm_body,
        grid=(B,),
        in_specs=[
            pl.BlockSpec((1, N, Cin), lambda b: (b, 0, 0)),
            pl.BlockSpec((Cout, Cin), lambda b: (0, 0)),
        ],
        out_specs=pl.BlockSpec((1, N, Cout), lambda b: (b, 0, 0)),
        out_shape=jax.ShapeDtypeStruct((B, N, Cout), jnp.float32),
    )(h, w)


def _sc_gather(table, idx):
    R, C = table.shape
    (M,) = idx.shape
    info = plsc.get_sparse_core_info()
    nw = info.num_cores * info.num_subcores
    m_per_w = M // nw
    ch = 256
    n_ch = m_per_w // ch
    assert m_per_w % ch == 0 and n_ch % 2 == 0
    mesh = plsc.VectorSubcoreMesh(core_axis_name="c", subcore_axis_name="s")

    @functools.partial(
        pl.kernel, mesh=mesh,
        out_type=jax.ShapeDtypeStruct((M, C), jnp.float32),
        scratch_types=[
            pltpu.VMEM((ch,), jnp.int32),
            pltpu.VMEM((ch,), jnp.int32),
            pltpu.VMEM((ch, C), jnp.float32),
            pltpu.VMEM((ch, C), jnp.float32),
            pltpu.SemaphoreType.DMA,
            pltpu.SemaphoreType.DMA,
        ],
    )
    def k(table_hbm, idx_hbm, out_hbm, idx0, idx1, rows0, rows1, s0, s1):
        wid = lax.axis_index("s") * info.num_cores + lax.axis_index("c")
        base = wid * m_per_w

        def issue(j, idx_v, rows_v, sem):
            off = base + j * ch
            pltpu.sync_copy(idx_hbm.at[pl.ds(off, ch)], idx_v)
            pltpu.async_copy(table_hbm.at[idx_v], rows_v, sem)

        def drain(j, idx_v, rows_v, sem):
            pltpu.make_async_copy(table_hbm.at[idx_v], rows_v, sem).wait()
            pltpu.sync_copy(rows_v, out_hbm.at[pl.ds(base + j * ch, ch)])

        issue(0, idx0, rows0, s0)

        def step2(jj, _):
            j = jj * 2
            issue(j + 1, idx1, rows1, s1)
            drain(j, idx0, rows0, s0)

            @pl.when(jj + 1 < n_ch // 2)
            def _():
                issue(j + 2, idx0, rows0, s0)

            drain(j + 1, idx1, rows1, s1)
            return 0

        lax.fori_loop(0, n_ch // 2, step2, 0)

    return k(table, idx)


def _acc_rows(out_ref, s, ss):
    rio = lax.broadcasted_iota(jnp.int32, out_ref.shape, 0)
    upd = jnp.where(rio == 0, jnp.broadcast_to(s, out_ref.shape), 0.0)
    upd = upd + jnp.where(rio == 1, jnp.broadcast_to(ss, out_ref.shape), 0.0)
    out_ref[...] += upd


def _h1pre(nbr_ref, z_ref, w1_ref):
    cv = z_ref.shape[-1]
    nb = nbr_ref[0, 0][:, :cv]
    ctr = jnp.broadcast_to(z_ref[0, 0][:, None, :], (NPB, KNB, cv))
    ctr = ctr.reshape(NPB * KNB, cv)
    d16 = _cast16(nb - ctr)
    c16 = _cast16(ctr)
    parts = [d16, c16]
    if 2 * cv < 128:
        parts.append(jnp.zeros((NPB * KNB, 128 - 2 * cv), d16.dtype))
    feat = jnp.concatenate(parts, axis=1)
    return _nt_bf16(feat, w1_ref[...])


def _ec_stats1_body(nbr_ref, z_ref, w1_ref, out_ref):
    @pl.when(jnp.logical_and(pl.program_id(0) == 0, pl.program_id(1) == 0))
    def _():
        out_ref[...] = jnp.zeros_like(out_ref)

    hh = _h1pre(nbr_ref, z_ref, w1_ref)
    s = jnp.sum(hh, axis=0, keepdims=True)
    ss = jnp.sum(hh * hh, axis=0, keepdims=True)
    _acc_rows(out_ref, s, ss)


def _ec_stats2_body(nbr_ref, z_ref, w1_ref, sc1_ref, w2_ref, out_ref):
    @pl.when(jnp.logical_and(pl.program_id(0) == 0, pl.program_id(1) == 0))
    def _():
        out_ref[...] = jnp.zeros_like(out_ref)

    hh = _h1pre(nbr_ref, z_ref, w1_ref)
    h1 = _lrelu(hh * sc1_ref[0:1, :] + sc1_ref[1:2, :])
    g = _nt_bf16(h1, w2_ref[...])
    s = jnp.sum(g, axis=0, keepdims=True)
    ss = jnp.sum(g * g, axis=0, keepdims=True)
    _acc_rows(out_ref, s, ss)


def _ec_apply_body(nbr_ref, z_ref, w1_ref, sc1_ref, w2_ref, sc2_ref, out_ref,
                   *, dup_out):
    c2 = w2_ref.shape[0]
    hh = _h1pre(nbr_ref, z_ref, w1_ref)
    h1 = _lrelu(hh * sc1_ref[0:1, :] + sc1_ref[1:2, :])
    g = _nt_bf16(h1, w2_ref[...])
    t = _lrelu(g * sc2_ref[0:1, :] + sc2_ref[1:2, :])
    m = jnp.max(t.reshape(NPB, KNB, c2), axis=1)
    if dup_out:
        m = jnp.concatenate([m, m], axis=1)
    out_ref[0, 0] = m


def _ec_grid_call(body, nbr4, z4, extras, out_shape, out_spec):
    B = z4.shape[0]
    c1 = z4.shape[-1]
    cg = nbr4.shape[-1]
    in_specs = [
        pl.BlockSpec((1, 1, NPB * KNB, cg), lambda b, r: (b, r, 0, 0)),
        pl.BlockSpec((1, 1, NPB, c1), lambda b, r: (b, r, 0, 0)),
    ]
    args = [nbr4, z4]
    for e in extras:
        in_specs.append(
            pl.BlockSpec(e.shape, (lambda nd: lambda b, r: (0,) * nd)(e.ndim)))
        args.append(e)
    return pl.pallas_call(
        body,
        grid=(B, NBLK),
        in_specs=in_specs,
        out_specs=out_spec,
        out_shape=out_shape,
    )(*args)


def _stats_to_scale(st, g, b, cnt):
    mean = st[0] / cnt
    var = st[1] / cnt - mean * mean
    s = g / jnp.sqrt(var + EPS)
    o = b - mean * s
    pad = jnp.zeros((6, s.shape[0]), jnp.float32)
    return jnp.concatenate([s[None, :], o[None, :], pad], axis=0)


def _edge_conv(xr, gidx_flat, w1p, w2, g1, b1, g2, b2, dup_out=False):
    B, N, cv = xr.shape
    c1 = w1p.shape[0]
    c2 = w2.shape[0]
    xp = xr if cv == 128 else jnp.pad(xr, ((0, 0), (0, 0), (0, 128 - cv)))
    nbr = _sc_gather(xp.reshape(B * N, 128), gidx_flat)
    nbr4 = nbr.reshape(B, NBLK, NPB * KNB, 128)
    z4 = xr.reshape(B, NBLK, NPB, cv)
    cnt = B * N * KNB
    st1 = _ec_grid_call(
        _ec_stats1_body, nbr4, z4, [w1p],
        jax.ShapeDtypeStruct((8, c1), jnp.float32),
        pl.BlockSpec((8, c1), lambda b, r: (0, 0)))
    sc1 = _stats_to_scale(st1, g1, b1, cnt)
    st2 = _ec_grid_call(
        _ec_stats2_body, nbr4, z4, [w1p, sc1, w2],
        jax.ShapeDtypeStruct((8, c2), jnp.float32),
        pl.BlockSpec((8, c2), lambda b, r: (0, 0)))
    sc2 = _stats_to_scale(st2, g2, b2, cnt)
    c_out = 2 * c2 if dup_out else c2
    out = _ec_grid_call(
        functools.partial(_ec_apply_body, dup_out=dup_out), nbr4, z4,
        [w1p, sc1, w2, sc2],
        jax.ShapeDtypeStruct((B, NBLK, NPB, c_out), jnp.float32),
        pl.BlockSpec((1, 1, NPB, c_out), lambda b, r: (b, r, 0, 0)))
    return out.reshape(B, N, c_out)


def _lin_stats_body(h_ref, w_ref, out_ref):
    @pl.when(pl.program_id(0) == 0)
    def _():
        out_ref[...] = jnp.zeros_like(out_ref)

    pre = _nt_bf16(h_ref[0], w_ref[...])
    s = jnp.sum(pre, axis=0, keepdims=True)
    ss = jnp.sum(pre * pre, axis=0, keepdims=True)
    _acc_rows(out_ref, s, ss)


def _lin_stats(h, w):
    B, N, cin = h.shape
    cout = w.shape[0]
    return pl.pallas_call(
        _lin_stats_body,
        grid=(B,),
        in_specs=[
            pl.BlockSpec((1, N, cin), lambda b: (b, 0, 0)),
            pl.BlockSpec((cout, cin), lambda b: (0, 0)),
        ],
        out_specs=pl.BlockSpec((8, cout), lambda b: (0, 0)),
        out_shape=jax.ShapeDtypeStruct((8, cout), jnp.float32),
    )(h, w)


def _lin_max_body(h_ref, w_ref, sc_ref, out_ref):
    b = pl.program_id(0)
    pre = _nt_bf16(h_ref[0], w_ref[...])
    act = _lrelu(pre * sc_ref[0:1, :] + sc_ref[1:2, :])
    row = jnp.max(act, axis=0, keepdims=True)
    rio = lax.broadcasted_iota(jnp.int32, out_ref.shape, 0)
    out_ref[...] = jnp.where(rio == b, jnp.broadcast_to(row, out_ref.shape),
                             out_ref[...])


def _lin_max(h, w, sc):
    B, N, cin = h.shape
    cout = w.shape[0]
    return pl.pallas_call(
        _lin_max_body,
        grid=(B,),
        in_specs=[
            pl.BlockSpec((1, N, cin), lambda b: (b, 0, 0)),
            pl.BlockSpec((cout, cin), lambda b: (0, 0)),
            pl.BlockSpec((8, cout), lambda b: (0, 0)),
        ],
        out_specs=pl.BlockSpec((B, cout), lambda b: (0, 0)),
        out_shape=jax.ShapeDtypeStruct((B, cout), jnp.float32),
    )(h, w, sc)


def _head_body(v_ref, w2_ref, gb2_ref, w3_ref, gb3_ref, w4_ref, b4_ref,
               cls_ref, xv_ref):
    def bn_batch(h, gb):
        m = jnp.mean(h, axis=0, keepdims=True)
        var = jnp.mean(h * h, axis=0, keepdims=True) - m * m
        return (h - m) / jnp.sqrt(var + EPS) * gb[0:1, :] + gb[1:2, :]

    h = _lrelu(bn_batch(_nt_bf16(v_ref[...], w2_ref[...]), gb2_ref))
    xv = _lrelu(bn_batch(_nt_bf16(h, w3_ref[...]) + gb3_ref[2:3, :], gb3_ref))
    xv_ref[...] = xv
    cls_ref[...] = _nt_bf16(xv, w4_ref[...]) + b4_ref[0:1, :]


def _head(vector, w2, g2, b2, w3, b3lin, g3, b3, w4, b4):
    B = vector.shape[0]
    gb2 = jnp.concatenate([g2[None], b2[None]], axis=0)
    gb3 = jnp.concatenate([g3[None], b3[None], b3lin[None]], axis=0)
    full = lambda shape: pl.BlockSpec(shape, lambda: tuple(0 for _ in shape))
    return pl.pallas_call(
        _head_body,
        in_specs=[full(vector.shape), full(w2.shape), full(gb2.shape),
                  full(w3.shape), full(gb3.shape), full(w4.shape),
                  full((1, b4.shape[0]))],
        out_specs=[full((B, w4.shape[0])), full((B, w3.shape[0]))],
        out_shape=[jax.ShapeDtypeStruct((B, w4.shape[0]), jnp.float32),
                   jax.ShapeDtypeStruct((B, w3.shape[0]), jnp.float32)],
    )(vector, w2, gb2, w3, gb3, w4, b4[None, :])


def _xv_row(xv_ref, wv_ref, b):
    crows = _nt_bf16(xv_ref[...], wv_ref[...])
    rio = lax.broadcasted_iota(jnp.int32, crows.shape, 0)
    return jnp.sum(jnp.where(rio == b, crows, 0.0), axis=0, keepdims=True)


def _seg_stats_body(h1_ref, h2_ref, xv_ref, wa_ref, wb_ref, wv_ref, out_ref):
    @pl.when(pl.program_id(0) == 0)
    def _():
        out_ref[...] = jnp.zeros_like(out_ref)

    crow = _xv_row(xv_ref, wv_ref, pl.program_id(0))
    pre = _nt_bf16(h1_ref[0], wa_ref[...]) + _nt_bf16(h2_ref[0], wb_ref[...]) + crow
    s = jnp.sum(pre, axis=0, keepdims=True)
    ss = jnp.sum(pre * pre, axis=0, keepdims=True)
    _acc_rows(out_ref, s, ss)


def _seg_apply_body(h1_ref, h2_ref, xv_ref, wa_ref, wb_ref, wv_ref, sc_ref,
                    out_ref):
    crow = _xv_row(xv_ref, wv_ref, pl.program_id(0))
    pre = _nt_bf16(h1_ref[0], wa_ref[...]) + _nt_bf16(h2_ref[0], wb_ref[...]) + crow
    out_ref[0] = _lrelu(pre * sc_ref[0:1, :] + sc_ref[1:2, :])


def _seg_call(body, h1, h2, xv, wa, wb, wv, sc=None):
    B, N, ca = h1.shape
    cout = wa.shape[0]
    in_specs = [
        pl.BlockSpec((1, N, ca), lambda b: (b, 0, 0)),
        pl.BlockSpec((1, N, h2.shape[2]), lambda b: (b, 0, 0)),
        pl.BlockSpec(xv.shape, lambda b: (0, 0)),
        pl.BlockSpec(wa.shape, lambda b: (0, 0)),
        pl.BlockSpec(wb.shape, lambda b: (0, 0)),
        pl.BlockSpec(wv.shape, lambda b: (0, 0)),
    ]
    args = [h1, h2, xv, wa, wb, wv]
    if sc is None:
        out_spec = pl.BlockSpec((8, cout), lambda b: (0, 0))
        out_shape = jax.ShapeDtypeStruct((8, cout), jnp.float32)
    else:
        in_specs.append(pl.BlockSpec((8, cout), lambda b: (0, 0)))
        args.append(sc)
        out_spec = pl.BlockSpec((1, N, cout), lambda b: (b, 0, 0))
        out_shape = jax.ShapeDtypeStruct((B, N, cout), jnp.float32)
    return pl.pallas_call(
        body,
        grid=(B,),
        in_specs=in_specs,
        out_specs=out_spec,
        out_shape=out_shape,
    )(*args)


def kernel(x, params):
    p = params
    B, _, N = x.shape
    xt = jnp.transpose(x, (0, 2, 1))
    xt8 = jnp.pad(xt, ((0, 0), (0, 0), (0, 5)))
    xt64 = jnp.pad(xt, ((0, 0), (0, 0), (0, 61)))

    w1 = p['ec0_w1']
    w1p0 = jnp.pad(jnp.concatenate([
        jnp.pad(w1[:, :3], ((0, 0), (0, 5))),
        jnp.pad(w1[:, 3:], ((0, 0), (0, 5)))], axis=1),
        ((0, 0), (0, 112)))
    gidx0 = _knn_topk(xt64).reshape(-1)
    x0cat = _edge_conv(xt8, gidx0, w1p0, p['ec0_w2'],
                       p['ec0_g1'], p['ec0_b1'], p['ec0_g2'], p['ec0_b2'],
                       dup_out=True)

    pn0_wp = jnp.pad(p['pn0_w'], ((0, 0), (0, 64)))
    st = _lin_stats(x0cat, pn0_wp)
    sc = _stats_to_scale(st, p['pn0_g'], p['pn0_b'], B * N)
    x_t0 = _lin_max(x0cat, pn0_wp, sc)

    gidx1 = _knn_topk(x0cat).reshape(-1)
    x1 = _edge_conv(x0cat, gidx1, p['ec1_w1'], p['ec1_w2'],
                    p['ec1_g1'], p['ec1_b1'], p['ec1_g2'], p['ec1_b2'])

    st = _lin_stats(x1, p['pn1_w'])
    sc = _stats_to_scale(st, p['pn1_g'], p['pn1_b'], B * N)
    x_t1 = _lin_max(x1, p['pn1_w'], sc)

    vector = jnp.concatenate([x_t0, x_t1], axis=1)
    logits_cls, x_vec = _head(vector, p['lin2_w'], p['bn2_g'], p['bn2_b'],
                              p['lin3_w'], p['lin3_b'], p['bn3_g'], p['bn3_b'],
                              p['lin4_w'], p['lin4_b'])

    w5 = p['pn5_w']
    w5v, w5a = w5[:, :256], w5[:, 256:]
    zb5 = jnp.zeros((128, 128), jnp.float32)
    st = _seg_call(_seg_stats_body, x1, x0cat, x_vec, w5a, zb5, w5v)
    sc5 = _stats_to_scale(st, p['pn5_g'], p['pn5_b'], B * N)
    h5 = _seg_call(_seg_apply_body, x1, x0cat, x_vec, w5a, zb5, w5v, sc5)

    w6 = p['pn6_w']
    w6a = w6[:, :128]
    w6bp = jnp.pad(w6[:, 128:], ((0, 0), (0, 64)))
    zv6 = jnp.zeros((128, x_vec.shape[1]), jnp.float32)
    st = _seg_call(_seg_stats_body, h5, x0cat, x_vec, w6a, w6bp, zv6)
    sc6 = _stats_to_scale(st, p['pn6_g'], p['pn6_b'], B * N)
    h6 = _seg_call(_seg_apply_body, h5, x0cat, x_vec, w6a, w6bp, zv6, sc6)

    w7 = jnp.pad(p['conv7_w'], ((0, 6), (0, 0)))
    seg8 = _bmm(h6, w7)
    logits_seg = jnp.transpose(seg8[:, :, :2], (0, 2, 1))

    return logits_cls, logits_seg, x, x
